# Initial kernel scaffold; baseline (speedup 1.0000x reference)
import jax
import jax.numpy as jnp
from jax.experimental import pallas as pl


def kernel(edge_index, text_f, vis_f, W_t, b_t, W_v, b_v, W_g0, b_g0, W_g1, b_g1, W_head, b_head):
    raise NotImplementedError("write your pallas kernel here")


# speedup vs baseline: 14.5839x; 14.5839x over previous
import functools

import jax
import jax.numpy as jnp
from jax import lax
from jax.experimental import pallas as pl
from jax.experimental.pallas import tpu as pltpu
from jax.experimental.pallas import tpu_sc as plsc

NC = 2
NS = 16
NW = NC * NS
CHUNK = 128


def _mesh():
    return plsc.VectorSubcoreMesh(
        core_axis_name="c", subcore_axis_name="s", num_cores=NC, num_subcores=NS
    )


def _fill(ref, n, value):
    def body(i, _):
        ref[pl.ds(i * 16, 16)] = jnp.full((16,), value, jnp.float32)
        return 0
    lax.fori_loop(0, n // 16, body, 0)


def _sc_degrees(src, dst, n_pad):
    E = src.shape[0]
    assert E % CHUNK == 0
    n_chunks = E // CHUNK
    rpt = n_pad // NS

    def body(src_hbm, dst_hbm, out_hbm, idx_v, ones_v, zer_v, din_sp, dout_sp):
        c = lax.axis_index("c")
        s = lax.axis_index("s")
        wid = c * NS + s

        _fill(zer_v, rpt, 0.0)
        _fill(ones_v, CHUNK, 1.0)
        pltpu.sync_copy(zer_v, din_sp.at[pl.ds(s * rpt, rpt)])
        pltpu.sync_copy(zer_v, dout_sp.at[pl.ds(s * rpt, rpt)])
        plsc.subcore_barrier()

        count = (n_chunks - wid + NW - 1) // NW

        def ebody(j, _):
            b = (j * NW + wid) * CHUNK
            pltpu.sync_copy(src_hbm.at[pl.ds(b, CHUNK)], idx_v)
            pltpu.sync_copy(ones_v, dout_sp.at[idx_v], add=True)
            pltpu.sync_copy(dst_hbm.at[pl.ds(b, CHUNK)], idx_v)
            pltpu.sync_copy(ones_v, din_sp.at[idx_v], add=True)
            return 0

        lax.fori_loop(0, count, ebody, 0)
        plsc.subcore_barrier()

        pltpu.sync_copy(din_sp.at[pl.ds(s * rpt, rpt)],
                        out_hbm.at[c, 0, pl.ds(s * rpt, rpt)])
        pltpu.sync_copy(dout_sp.at[pl.ds(s * rpt, rpt)],
                        out_hbm.at[c, 1, pl.ds(s * rpt, rpt)])

    f = pl.kernel(
        body,
        out_type=jax.ShapeDtypeStruct((NC, 2, n_pad), jnp.float32),
        mesh=_mesh(),
        scratch_types=[
            pltpu.VMEM((CHUNK,), jnp.int32),
            pltpu.VMEM((CHUNK,), jnp.float32),
            pltpu.VMEM((rpt,), jnp.float32),
            pltpu.VMEM_SHARED((n_pad,), jnp.float32),
            pltpu.VMEM_SHARED((n_pad,), jnp.float32),
        ],
    )
    return f(src, dst)


def _sc_aggregate(src, dst, p, n_pad):
    E = src.shape[0]
    H = p.shape[1]
    assert E % CHUNK == 0
    n_chunks = E // CHUNK
    rpt = n_pad // NS
    ZR = 64
    assert rpt % ZR == 0

    def body(src_hbm, dst_hbm, p_hbm, out_hbm, idxs_v, idxd_v, rows_v, zer_v, acc_sp, sem):
        c = lax.axis_index("c")
        s = lax.axis_index("s")
        wid = c * NS + s

        def zfill(i, _):
            zer_v[i, pl.ds(0, 16)] = jnp.zeros((16,), jnp.float32)
            zer_v[i, pl.ds(16, 16)] = jnp.zeros((16,), jnp.float32)
            zer_v[i, pl.ds(32, 16)] = jnp.zeros((16,), jnp.float32)
            zer_v[i, pl.ds(48, 16)] = jnp.zeros((16,), jnp.float32)
            return 0
        lax.fori_loop(0, ZR, zfill, 0)

        def zcopy(i, _):
            pltpu.sync_copy(zer_v, acc_sp.at[pl.ds(s * rpt + i * ZR, ZR)])
            return 0
        lax.fori_loop(0, rpt // ZR, zcopy, 0)
        plsc.subcore_barrier()

        count = (n_chunks - wid + NW - 1) // NW

        def ebody(j, _):
            b = (j * NW + wid) * CHUNK
            pltpu.sync_copy(src_hbm.at[pl.ds(b, CHUNK)], idxs_v)
            pltpu.async_copy(p_hbm.at[idxs_v], rows_v, sem).wait()
            pltpu.sync_copy(dst_hbm.at[pl.ds(b, CHUNK)], idxd_v)
            pltpu.sync_copy(rows_v, acc_sp.at[idxd_v], add=True)
            return 0

        lax.fori_loop(0, count, ebody, 0)
        plsc.subcore_barrier()

        pltpu.sync_copy(acc_sp.at[pl.ds(s * rpt, rpt)],
                        out_hbm.at[c, pl.ds(s * rpt, rpt)])

    f = pl.kernel(
        body,
        out_type=jax.ShapeDtypeStruct((NC, n_pad, H), jnp.float32),
        mesh=_mesh(),
        scratch_types=[
            pltpu.VMEM((CHUNK,), jnp.int32),
            pltpu.VMEM((CHUNK,), jnp.int32),
            pltpu.VMEM((CHUNK, H), jnp.float32),
            pltpu.VMEM((64, H), jnp.float32),
            pltpu.VMEM_SHARED((n_pad, H), jnp.float32),
            pltpu.SemaphoreType.DMA,
        ],
        compiler_params=pltpu.CompilerParams(use_tc_tiling_on_sc=False),
    )
    return f(src, dst, p)


def _tc_encode(text_f, vis_f, W_t, b_t, W_v, b_v, W_g0, dout):
    N, T = text_f.shape
    V = vis_f.shape[1]
    H = W_t.shape[1]
    RB = 1000
    assert N % RB == 0

    def body(t_ref, v_ref, wt_ref, bt_ref, wv_ref, bv_ref, wg_ref, do_ref, o_ref):
        ht = jnp.maximum(
            jnp.dot(t_ref[...], wt_ref[...], preferred_element_type=jnp.float32)
            + bt_ref[...], 0.0)
        hv = jnp.maximum(
            jnp.dot(v_ref[...], wv_ref[...], preferred_element_type=jnp.float32)
            + bv_ref[...], 0.0)
        h = jnp.concatenate([ht, hv], axis=1)
        z = jnp.dot(h, wg_ref[...], preferred_element_type=jnp.float32)
        d = do_ref[0] + do_ref[1]
        inv = 1.0 / jnp.sqrt(jnp.maximum(d, 1.0))
        o_ref[...] = z * inv

    return pl.pallas_call(
        body,
        grid=(N // RB,),
        in_specs=[
            pl.BlockSpec((RB, T), lambda i: (i, 0)),
            pl.BlockSpec((RB, V), lambda i: (i, 0)),
            pl.BlockSpec((T, H), lambda i: (0, 0)),
            pl.BlockSpec((1, H), lambda i: (0, 0)),
            pl.BlockSpec((V, H), lambda i: (0, 0)),
            pl.BlockSpec((1, H), lambda i: (0, 0)),
            pl.BlockSpec((2 * H, H), lambda i: (0, 0)),
            pl.BlockSpec((NC, RB, 1), lambda i: (0, i, 0)),
        ],
        out_specs=pl.BlockSpec((RB, H), lambda i: (i, 0)),
        out_shape=jax.ShapeDtypeStruct((N, H), jnp.float32),
    )(text_f, vis_f, W_t, b_t, W_v, b_v, W_g0, dout)


def _tc_mid(agg, din, dout, b_g0, W_g1, N):
    H = W_g1.shape[0]
    RB = 1000
    assert N % RB == 0

    def body(a_ref, di_ref, do_ref, b_ref, w_ref, o_ref):
        a = a_ref[0] + a_ref[1]
        inv_in = 1.0 / jnp.sqrt(jnp.maximum(di_ref[0] + di_ref[1], 1.0))
        h = jnp.maximum(a * inv_in + b_ref[...], 0.0)
        z = jnp.dot(h, w_ref[...], preferred_element_type=jnp.float32)
        inv_out = 1.0 / jnp.sqrt(jnp.maximum(do_ref[0] + do_ref[1], 1.0))
        o_ref[...] = z * inv_out

    return pl.pallas_call(
        body,
        grid=(N // RB,),
        in_specs=[
            pl.BlockSpec((NC, RB, H), lambda i: (0, i, 0)),
            pl.BlockSpec((NC, RB, 1), lambda i: (0, i, 0)),
            pl.BlockSpec((NC, RB, 1), lambda i: (0, i, 0)),
            pl.BlockSpec((1, H), lambda i: (0, 0)),
            pl.BlockSpec((H, H), lambda i: (0, 0)),
        ],
        out_specs=pl.BlockSpec((RB, H), lambda i: (i, 0)),
        out_shape=jax.ShapeDtypeStruct((N, H), jnp.float32),
    )(agg, din, dout, b_g0, W_g1)


def _tc_head(agg, din, b_g1, W_head, b_head, N):
    H, C = W_head.shape
    RB = 1000
    assert N % RB == 0

    def body(a_ref, di_ref, b_ref, w_ref, bh_ref, o_ref):
        a = a_ref[0] + a_ref[1]
        inv_in = 1.0 / jnp.sqrt(jnp.maximum(di_ref[0] + di_ref[1], 1.0))
        h = jnp.maximum(a * inv_in + b_ref[...], 0.0)
        o_ref[...] = (
            jnp.dot(h, w_ref[...], preferred_element_type=jnp.float32) + bh_ref[...]
        )

    return pl.pallas_call(
        body,
        grid=(N // RB,),
        in_specs=[
            pl.BlockSpec((NC, RB, H), lambda i: (0, i, 0)),
            pl.BlockSpec((NC, RB, 1), lambda i: (0, i, 0)),
            pl.BlockSpec((1, H), lambda i: (0, 0)),
            pl.BlockSpec((H, C), lambda i: (0, 0)),
            pl.BlockSpec((1, C), lambda i: (0, 0)),
        ],
        out_specs=pl.BlockSpec((RB, C), lambda i: (i, 0)),
        out_shape=jax.ShapeDtypeStruct((N, C), jnp.float32),
    )(agg, din, b_g1, W_head, b_head)


def kernel(edge_index, text_f, vis_f, W_t, b_t, W_v, b_v, W_g0, b_g0, W_g1, b_g1,
           W_head, b_head):
    N = text_f.shape[0]
    H = W_t.shape[1]
    n_pad = -(-N // (NS * 64)) * (NS * 64)

    src = edge_index[0]
    dst = edge_index[1]

    deg = _sc_degrees(src, dst, n_pad)
    din = deg[:, 0, :, None]
    dout = deg[:, 1, :, None]

    p0 = _tc_encode(text_f, vis_f, W_t, b_t.reshape(1, -1), W_v,
                    b_v.reshape(1, -1), W_g0, dout)
    agg0 = _sc_aggregate(src, dst, p0, n_pad)
    p1 = _tc_mid(agg0, din, dout, b_g0.reshape(1, -1), W_g1, N)
    agg1 = _sc_aggregate(src, dst, p1, n_pad)
    out = _tc_head(agg1, din, b_g1.reshape(1, -1), W_head, b_head.reshape(1, -1), N)
    return out


# speedup vs baseline: 26.7678x; 1.8354x over previous
import functools

import jax
import jax.numpy as jnp
from jax import lax
from jax.experimental import pallas as pl
from jax.experimental.pallas import tpu as pltpu
from jax.experimental.pallas import tpu_sc as plsc

NC = 2
NS = 16
NW = NC * NS
LANE = 128


def _mesh():
    return plsc.VectorSubcoreMesh(
        core_axis_name="c", subcore_axis_name="s", num_cores=NC, num_subcores=NS
    )


def _fill(ref, n, value):
    def body(i, _):
        ref[pl.ds(i * 16, 16)] = jnp.full((16,), value, jnp.float32)
        return 0
    lax.fori_loop(0, n // 16, body, 0)


def _sc_degrees(eflat, n_pad):
    E = eflat.shape[1]
    CH = 128
    assert E % NW == 0
    per_tile = E // NW
    full = per_tile // CH
    tail = per_tile % CH
    pairs, odd = divmod(full, 2)
    assert tail % 16 == 0 and per_tile % 8 == 0
    rpt = n_pad // NS

    def body(e_hbm, out_hbm, sbA, dbA, sbB, dbB, sbt, dbt, ones_v, onest_v, zer_v,
             din_sp, dout_sp, semA, semB, semC, semD):
        c = lax.axis_index("c")
        s = lax.axis_index("s")
        wid = c * NS + s
        base0 = wid * per_tile

        _fill(zer_v, rpt, 0.0)
        _fill(ones_v, CH, 1.0)
        if tail:
            _fill(onest_v, tail, 1.0)
        pltpu.sync_copy(zer_v, din_sp.at[pl.ds(s * rpt, rpt)])
        pltpu.sync_copy(zer_v, dout_sp.at[pl.ds(s * rpt, rpt)])
        plsc.subcore_barrier()

        def pbody(i, _):
            b0 = base0 + (2 * i) * CH
            b1 = b0 + CH
            dA = pltpu.async_copy(e_hbm.at[0, pl.ds(b0, CH)], sbA, semA)
            dB = pltpu.async_copy(e_hbm.at[1, pl.ds(b0, CH)], dbA, semB)
            dC = pltpu.async_copy(e_hbm.at[0, pl.ds(b1, CH)], sbB, semC)
            dD = pltpu.async_copy(e_hbm.at[1, pl.ds(b1, CH)], dbB, semD)
            dA.wait()
            pltpu.sync_copy(ones_v, dout_sp.at[sbA], add=True)
            dB.wait()
            pltpu.sync_copy(ones_v, din_sp.at[dbA], add=True)
            dC.wait()
            pltpu.sync_copy(ones_v, dout_sp.at[sbB], add=True)
            dD.wait()
            pltpu.sync_copy(ones_v, din_sp.at[dbB], add=True)
            return 0

        lax.fori_loop(0, pairs, pbody, 0)

        if odd:
            b0 = base0 + (pairs * 2) * CH
            pltpu.sync_copy(e_hbm.at[0, pl.ds(b0, CH)], sbA)
            pltpu.sync_copy(ones_v, dout_sp.at[sbA], add=True)
            pltpu.sync_copy(e_hbm.at[1, pl.ds(b0, CH)], dbA)
            pltpu.sync_copy(ones_v, din_sp.at[dbA], add=True)

        if tail:
            bt = base0 + full * CH
            pltpu.sync_copy(e_hbm.at[0, pl.ds(bt, tail)], sbt)
            pltpu.sync_copy(onest_v, dout_sp.at[sbt], add=True)
            pltpu.sync_copy(e_hbm.at[1, pl.ds(bt, tail)], dbt)
            pltpu.sync_copy(onest_v, din_sp.at[dbt], add=True)

        plsc.subcore_barrier()

        pltpu.sync_copy(din_sp.at[pl.ds(s * rpt, rpt)],
                        out_hbm.at[c, 0, pl.ds(s * rpt, rpt)])
        pltpu.sync_copy(dout_sp.at[pl.ds(s * rpt, rpt)],
                        out_hbm.at[c, 1, pl.ds(s * rpt, rpt)])

    f = pl.kernel(
        body,
        out_type=jax.ShapeDtypeStruct((NC, 2, n_pad), jnp.float32),
        mesh=_mesh(),
        scratch_types=[
            pltpu.VMEM((CH,), jnp.int32),
            pltpu.VMEM((CH,), jnp.int32),
            pltpu.VMEM((CH,), jnp.int32),
            pltpu.VMEM((CH,), jnp.int32),
            pltpu.VMEM((max(tail, 16),), jnp.int32),
            pltpu.VMEM((max(tail, 16),), jnp.int32),
            pltpu.VMEM((CH,), jnp.float32),
            pltpu.VMEM((max(tail, 16),), jnp.float32),
            pltpu.VMEM((rpt,), jnp.float32),
            pltpu.VMEM_SHARED((n_pad,), jnp.float32),
            pltpu.VMEM_SHARED((n_pad,), jnp.float32),
            pltpu.SemaphoreType.DMA,
            pltpu.SemaphoreType.DMA,
            pltpu.SemaphoreType.DMA,
            pltpu.SemaphoreType.DMA,
        ],
        compiler_params=pltpu.CompilerParams(use_tc_tiling_on_sc=False),
    )
    return f(eflat)


def _sc_aggregate(eflat, p, n_pad):
    E = eflat.shape[1]
    H = p.shape[1]
    CH = 128
    assert E % NW == 0
    per_tile = E // NW
    full = per_tile // CH
    tail = per_tile % CH
    pairs, odd = divmod(full, 2)
    assert tail % 8 == 0 and per_tile % 8 == 0
    rpt = n_pad // NS
    ZR = 64
    assert rpt % ZR == 0

    def body(e_hbm, p_hbm, out_hbm, sbig, dbA, dbB, dbt, rowsA, rowsB, rowst,
             zer_v, acc_sp, semDA, semDB, semGA, semGB, semSA, semSB):
        c = lax.axis_index("c")
        s = lax.axis_index("s")
        wid = c * NS + s
        base0 = wid * per_tile

        def zfill(i, _):
            zer_v[i, pl.ds(0, 16)] = jnp.zeros((16,), jnp.float32)
            zer_v[i, pl.ds(16, 16)] = jnp.zeros((16,), jnp.float32)
            zer_v[i, pl.ds(32, 16)] = jnp.zeros((16,), jnp.float32)
            zer_v[i, pl.ds(48, 16)] = jnp.zeros((16,), jnp.float32)
            return 0
        lax.fori_loop(0, ZR, zfill, 0)

        def zcopy(i, _):
            pltpu.sync_copy(zer_v, acc_sp.at[pl.ds(s * rpt + i * ZR, ZR)])
            return 0
        lax.fori_loop(0, rpt // ZR, zcopy, 0)

        pltpu.sync_copy(e_hbm.at[0, pl.ds(base0, per_tile)], sbig)
        plsc.subcore_barrier()

        def pbody(i, _):
            b0 = (2 * i) * CH
            b1 = b0 + CH
            dA = pltpu.async_copy(e_hbm.at[1, pl.ds(base0 + b0, CH)], dbA, semDA)
            gA = pltpu.async_copy(p_hbm.at[sbig.at[pl.ds(b0, CH)]], rowsA, semGA)
            dB = pltpu.async_copy(e_hbm.at[1, pl.ds(base0 + b1, CH)], dbB, semDB)
            gB = pltpu.async_copy(p_hbm.at[sbig.at[pl.ds(b1, CH)]], rowsB, semGB)
            dA.wait()
            gA.wait()
            sA = pltpu.async_copy(rowsA, acc_sp.at[dbA], semSA, add=True)
            dB.wait()
            gB.wait()
            sB = pltpu.async_copy(rowsB, acc_sp.at[dbB], semSB, add=True)
            sA.wait()
            sB.wait()
            return 0

        lax.fori_loop(0, pairs, pbody, 0)

        if odd:
            b0 = (pairs * 2) * CH
            pltpu.sync_copy(e_hbm.at[1, pl.ds(base0 + b0, CH)], dbA)
            pltpu.async_copy(p_hbm.at[sbig.at[pl.ds(b0, CH)]], rowsA, semGA).wait()
            pltpu.sync_copy(rowsA, acc_sp.at[dbA], add=True)

        if tail:
            bt = full * CH
            pltpu.sync_copy(e_hbm.at[1, pl.ds(base0 + bt, tail)], dbt)
            pltpu.async_copy(p_hbm.at[sbig.at[pl.ds(bt, tail)]], rowst, semGA).wait()
            pltpu.sync_copy(rowst, acc_sp.at[dbt], add=True)

        plsc.subcore_barrier()
        pltpu.sync_copy(acc_sp.at[pl.ds(s * rpt, rpt)],
                        out_hbm.at[c, pl.ds(s * rpt, rpt)])

    f = pl.kernel(
        body,
        out_type=jax.ShapeDtypeStruct((NC, n_pad, H), jnp.float32),
        mesh=_mesh(),
        scratch_types=[
            pltpu.VMEM((per_tile,), jnp.int32),
            pltpu.VMEM((CH,), jnp.int32),
            pltpu.VMEM((CH,), jnp.int32),
            pltpu.VMEM((max(tail, 8),), jnp.int32),
            pltpu.VMEM((CH, H), jnp.float32),
            pltpu.VMEM((CH, H), jnp.float32),
            pltpu.VMEM((max(tail, 8), H), jnp.float32),
            pltpu.VMEM((ZR, H), jnp.float32),
            pltpu.VMEM_SHARED((n_pad, H), jnp.float32),
            pltpu.SemaphoreType.DMA,
            pltpu.SemaphoreType.DMA,
            pltpu.SemaphoreType.DMA,
            pltpu.SemaphoreType.DMA,
            pltpu.SemaphoreType.DMA,
            pltpu.SemaphoreType.DMA,
        ],
        compiler_params=pltpu.CompilerParams(use_tc_tiling_on_sc=False),
    )
    return f(eflat, p)


def _tc_encode(text_f, vis_f, W_t, b_t, W_v, b_v, W_g0, dout):
    N, T = text_f.shape
    V = vis_f.shape[1]
    H = W_t.shape[1]
    RB = 1000
    assert N % RB == 0

    def body(t_ref, v_ref, wt_ref, bt_ref, wv_ref, bv_ref, wg_ref, do_ref, o_ref):
        ht = jnp.maximum(
            jnp.dot(t_ref[...], wt_ref[...], preferred_element_type=jnp.float32)
            + bt_ref[...], 0.0)
        hv = jnp.maximum(
            jnp.dot(v_ref[...], wv_ref[...], preferred_element_type=jnp.float32)
            + bv_ref[...], 0.0)
        h = jnp.concatenate([ht, hv], axis=1)
        z = jnp.dot(h, wg_ref[...], preferred_element_type=jnp.float32)
        d = do_ref[0] + do_ref[1]
        inv = 1.0 / jnp.sqrt(jnp.maximum(d, 1.0))
        o_ref[...] = z * inv

    return pl.pallas_call(
        body,
        grid=(N // RB,),
        in_specs=[
            pl.BlockSpec((RB, T), lambda i: (i, 0)),
            pl.BlockSpec((RB, V), lambda i: (i, 0)),
            pl.BlockSpec((T, H), lambda i: (0, 0)),
            pl.BlockSpec((1, H), lambda i: (0, 0)),
            pl.BlockSpec((V, H), lambda i: (0, 0)),
            pl.BlockSpec((1, H), lambda i: (0, 0)),
            pl.BlockSpec((2 * H, H), lambda i: (0, 0)),
            pl.BlockSpec((NC, RB, 1), lambda i: (0, i, 0)),
        ],
        out_specs=pl.BlockSpec((RB, H), lambda i: (i, 0)),
        out_shape=jax.ShapeDtypeStruct((N, H), jnp.float32),
    )(text_f, vis_f, W_t, b_t, W_v, b_v, W_g0, dout)


def _tc_mid(agg, din, dout, b_g0, W_g1, N):
    H = W_g1.shape[0]
    RB = 1000
    assert N % RB == 0

    def body(a_ref, di_ref, do_ref, b_ref, w_ref, o_ref):
        a = a_ref[0] + a_ref[1]
        inv_in = 1.0 / jnp.sqrt(jnp.maximum(di_ref[0] + di_ref[1], 1.0))
        h = jnp.maximum(a * inv_in + b_ref[...], 0.0)
        z = jnp.dot(h, w_ref[...], preferred_element_type=jnp.float32)
        inv_out = 1.0 / jnp.sqrt(jnp.maximum(do_ref[0] + do_ref[1], 1.0))
        o_ref[...] = z * inv_out

    return pl.pallas_call(
        body,
        grid=(N // RB,),
        in_specs=[
            pl.BlockSpec((NC, RB, H), lambda i: (0, i, 0)),
            pl.BlockSpec((NC, RB, 1), lambda i: (0, i, 0)),
            pl.BlockSpec((NC, RB, 1), lambda i: (0, i, 0)),
            pl.BlockSpec((1, H), lambda i: (0, 0)),
            pl.BlockSpec((H, H), lambda i: (0, 0)),
        ],
        out_specs=pl.BlockSpec((RB, H), lambda i: (i, 0)),
        out_shape=jax.ShapeDtypeStruct((N, H), jnp.float32),
    )(agg, din, dout, b_g0, W_g1)


def _tc_head(agg, din, b_g1, W_head, b_head, N):
    H, C = W_head.shape
    RB = 1000
    assert N % RB == 0

    def body(a_ref, di_ref, b_ref, w_ref, bh_ref, o_ref):
        a = a_ref[0] + a_ref[1]
        inv_in = 1.0 / jnp.sqrt(jnp.maximum(di_ref[0] + di_ref[1], 1.0))
        h = jnp.maximum(a * inv_in + b_ref[...], 0.0)
        o_ref[...] = (
            jnp.dot(h, w_ref[...], preferred_element_type=jnp.float32) + bh_ref[...]
        )

    return pl.pallas_call(
        body,
        grid=(N // RB,),
        in_specs=[
            pl.BlockSpec((NC, RB, H), lambda i: (0, i, 0)),
            pl.BlockSpec((NC, RB, 1), lambda i: (0, i, 0)),
            pl.BlockSpec((1, H), lambda i: (0, 0)),
            pl.BlockSpec((H, C), lambda i: (0, 0)),
            pl.BlockSpec((1, C), lambda i: (0, 0)),
        ],
        out_specs=pl.BlockSpec((RB, C), lambda i: (i, 0)),
        out_shape=jax.ShapeDtypeStruct((N, C), jnp.float32),
    )(agg, din, b_g1, W_head, b_head)


def kernel(edge_index, text_f, vis_f, W_t, b_t, W_v, b_v, W_g0, b_g0, W_g1, b_g1,
           W_head, b_head):
    N = text_f.shape[0]
    E = edge_index.shape[1]
    n_pad = -(-N // (NS * 64)) * (NS * 64)

    deg = _sc_degrees(edge_index, n_pad)
    din = deg[:, 0, :, None]
    dout = deg[:, 1, :, None]

    p0 = _tc_encode(text_f, vis_f, W_t, b_t.reshape(1, -1), W_v,
                    b_v.reshape(1, -1), W_g0, dout)
    agg0 = _sc_aggregate(edge_index, p0, n_pad)
    p1 = _tc_mid(agg0, din, dout, b_g0.reshape(1, -1), W_g1, N)
    agg1 = _sc_aggregate(edge_index, p1, n_pad)
    out = _tc_head(agg1, din, b_g1.reshape(1, -1), W_head, b_head.reshape(1, -1), N)
    return out


# speedup vs baseline: 30.5872x; 1.1427x over previous
import functools

import jax
import jax.numpy as jnp
from jax import lax
from jax.experimental import pallas as pl
from jax.experimental.pallas import tpu as pltpu
from jax.experimental.pallas import tpu_sc as plsc

NC = 2
NS = 16
NW = NC * NS
LANE = 128


def _mesh():
    return plsc.VectorSubcoreMesh(
        core_axis_name="c", subcore_axis_name="s", num_cores=NC, num_subcores=NS
    )


def _fill(ref, n, value):
    def body(i, _):
        ref[pl.ds(i * 16, 16)] = jnp.full((16,), value, jnp.float32)
        return 0
    lax.fori_loop(0, n // 16, body, 0)


def _sc_degrees(eflat, n_pad):
    E = eflat.shape[1]
    CH = 128
    assert E % NW == 0
    per_tile = E // NW
    full = per_tile // CH
    tail = per_tile % CH
    pairs, odd = divmod(full, 2)
    assert tail % 16 == 0 and per_tile % 8 == 0
    rpt = n_pad // NS

    def body(e_hbm, out_hbm, sbA, dbA, sbB, dbB, sbt, dbt, ones_v, onest_v, zer_v,
             din_sp, dout_sp, semA, semB, semC, semD):
        c = lax.axis_index("c")
        s = lax.axis_index("s")
        wid = c * NS + s
        base0 = wid * per_tile

        _fill(zer_v, rpt, 0.0)
        _fill(ones_v, CH, 1.0)
        if tail:
            _fill(onest_v, tail, 1.0)
        pltpu.sync_copy(zer_v, din_sp.at[pl.ds(s * rpt, rpt)])
        pltpu.sync_copy(zer_v, dout_sp.at[pl.ds(s * rpt, rpt)])
        plsc.subcore_barrier()

        def pbody(i, _):
            b0 = base0 + (2 * i) * CH
            b1 = b0 + CH
            dA = pltpu.async_copy(e_hbm.at[0, pl.ds(b0, CH)], sbA, semA)
            dB = pltpu.async_copy(e_hbm.at[1, pl.ds(b0, CH)], dbA, semB)
            dC = pltpu.async_copy(e_hbm.at[0, pl.ds(b1, CH)], sbB, semC)
            dD = pltpu.async_copy(e_hbm.at[1, pl.ds(b1, CH)], dbB, semD)
            dA.wait()
            pltpu.sync_copy(ones_v, dout_sp.at[sbA], add=True)
            dB.wait()
            pltpu.sync_copy(ones_v, din_sp.at[dbA], add=True)
            dC.wait()
            pltpu.sync_copy(ones_v, dout_sp.at[sbB], add=True)
            dD.wait()
            pltpu.sync_copy(ones_v, din_sp.at[dbB], add=True)
            return 0

        lax.fori_loop(0, pairs, pbody, 0)

        if odd:
            b0 = base0 + (pairs * 2) * CH
            pltpu.sync_copy(e_hbm.at[0, pl.ds(b0, CH)], sbA)
            pltpu.sync_copy(ones_v, dout_sp.at[sbA], add=True)
            pltpu.sync_copy(e_hbm.at[1, pl.ds(b0, CH)], dbA)
            pltpu.sync_copy(ones_v, din_sp.at[dbA], add=True)

        if tail:
            bt = base0 + full * CH
            pltpu.sync_copy(e_hbm.at[0, pl.ds(bt, tail)], sbt)
            pltpu.sync_copy(onest_v, dout_sp.at[sbt], add=True)
            pltpu.sync_copy(e_hbm.at[1, pl.ds(bt, tail)], dbt)
            pltpu.sync_copy(onest_v, din_sp.at[dbt], add=True)

        plsc.subcore_barrier()

        pltpu.sync_copy(din_sp.at[pl.ds(s * rpt, rpt)],
                        out_hbm.at[c, 0, pl.ds(s * rpt, rpt)])
        pltpu.sync_copy(dout_sp.at[pl.ds(s * rpt, rpt)],
                        out_hbm.at[c, 1, pl.ds(s * rpt, rpt)])

    f = pl.kernel(
        body,
        out_type=jax.ShapeDtypeStruct((NC, 2, n_pad), jnp.float32),
        mesh=_mesh(),
        scratch_types=[
            pltpu.VMEM((CH,), jnp.int32),
            pltpu.VMEM((CH,), jnp.int32),
            pltpu.VMEM((CH,), jnp.int32),
            pltpu.VMEM((CH,), jnp.int32),
            pltpu.VMEM((max(tail, 16),), jnp.int32),
            pltpu.VMEM((max(tail, 16),), jnp.int32),
            pltpu.VMEM((CH,), jnp.float32),
            pltpu.VMEM((max(tail, 16),), jnp.float32),
            pltpu.VMEM((rpt,), jnp.float32),
            pltpu.VMEM_SHARED((n_pad,), jnp.float32),
            pltpu.VMEM_SHARED((n_pad,), jnp.float32),
            pltpu.SemaphoreType.DMA,
            pltpu.SemaphoreType.DMA,
            pltpu.SemaphoreType.DMA,
            pltpu.SemaphoreType.DMA,
        ],
        compiler_params=pltpu.CompilerParams(use_tc_tiling_on_sc=False),
    )
    return f(eflat)


def _sc_aggregate(eflat, p, n_pad):
    E = eflat.shape[1]
    H = p.shape[1]
    CH = 128
    assert E % NW == 0
    per_tile = E // NW
    full = per_tile // CH
    tail = per_tile % CH
    triples, rem = divmod(full, 3)
    assert tail % 8 == 0 and per_tile % 8 == 0
    rpt = n_pad // NS
    ZR = 64
    assert rpt % ZR == 0

    def body(e_hbm, p_hbm, out_hbm, sbig, dbA, dbB, dbC, dbt, rowsA, rowsB,
             rowsC, rowst, zer_v, acc_sp, semDA, semDB, semDC, semGA, semGB,
             semGC, semSA, semSB, semSC):
        c = lax.axis_index("c")
        s = lax.axis_index("s")
        wid = c * NS + s
        base0 = wid * per_tile

        def zfill(i, _):
            zer_v[i, pl.ds(0, 16)] = jnp.zeros((16,), jnp.float32)
            zer_v[i, pl.ds(16, 16)] = jnp.zeros((16,), jnp.float32)
            zer_v[i, pl.ds(32, 16)] = jnp.zeros((16,), jnp.float32)
            zer_v[i, pl.ds(48, 16)] = jnp.zeros((16,), jnp.float32)
            return 0
        lax.fori_loop(0, ZR, zfill, 0)

        def zcopy(i, _):
            pltpu.sync_copy(zer_v, acc_sp.at[pl.ds(s * rpt + i * ZR, ZR)])
            return 0
        lax.fori_loop(0, rpt // ZR, zcopy, 0)

        pltpu.sync_copy(e_hbm.at[0, pl.ds(base0, per_tile)], sbig)
        plsc.subcore_barrier()

        bufs = ((dbA, rowsA, semDA, semGA, semSA),
                (dbB, rowsB, semDB, semGB, semSB),
                (dbC, rowsC, semDC, semGC, semSC))

        def tbody(i, _):
            descs = []
            for k, (db, rows, semD, semG, semS) in enumerate(bufs):
                b0 = (3 * i + k) * CH
                @pl.when(i > 0)
                def _(db=db, rows=rows, semS=semS):
                    pltpu.make_async_copy(rows, acc_sp.at[db], semS).wait()
                descs.append((
                    pltpu.async_copy(e_hbm.at[1, pl.ds(base0 + b0, CH)], db, semD),
                    pltpu.async_copy(p_hbm.at[sbig.at[pl.ds(b0, CH)]], rows, semG),
                ))
            for (d, g), (db, rows, _, _, semS) in zip(descs, bufs):
                d.wait()
                g.wait()
                pltpu.async_copy(rows, acc_sp.at[db], semS, add=True)
            return 0

        lax.fori_loop(0, triples, tbody, 0)
        if triples > 0:
            for db, rows, _, _, semS in bufs:
                pltpu.make_async_copy(rows, acc_sp.at[db], semS).wait()

        for r in range(rem):
            b0 = (triples * 3 + r) * CH
            pltpu.sync_copy(e_hbm.at[1, pl.ds(base0 + b0, CH)], dbA)
            pltpu.async_copy(p_hbm.at[sbig.at[pl.ds(b0, CH)]], rowsA, semGA).wait()
            pltpu.sync_copy(rowsA, acc_sp.at[dbA], add=True)

        if tail:
            bt = full * CH
            pltpu.sync_copy(e_hbm.at[1, pl.ds(base0 + bt, tail)], dbt)
            pltpu.async_copy(p_hbm.at[sbig.at[pl.ds(bt, tail)]], rowst, semGA).wait()
            pltpu.sync_copy(rowst, acc_sp.at[dbt], add=True)

        plsc.subcore_barrier()
        pltpu.sync_copy(acc_sp.at[pl.ds(s * rpt, rpt)],
                        out_hbm.at[c, pl.ds(s * rpt, rpt)])

    f = pl.kernel(
        body,
        out_type=jax.ShapeDtypeStruct((NC, n_pad, H), jnp.float32),
        mesh=_mesh(),
        scratch_types=[
            pltpu.VMEM((per_tile,), jnp.int32),
            pltpu.VMEM((CH,), jnp.int32),
            pltpu.VMEM((CH,), jnp.int32),
            pltpu.VMEM((CH,), jnp.int32),
            pltpu.VMEM((max(tail, 8),), jnp.int32),
            pltpu.VMEM((CH, H), jnp.float32),
            pltpu.VMEM((CH, H), jnp.float32),
            pltpu.VMEM((CH, H), jnp.float32),
            pltpu.VMEM((max(tail, 8), H), jnp.float32),
            pltpu.VMEM((ZR, H), jnp.float32),
            pltpu.VMEM_SHARED((n_pad, H), jnp.float32),
            pltpu.SemaphoreType.DMA,
            pltpu.SemaphoreType.DMA,
            pltpu.SemaphoreType.DMA,
            pltpu.SemaphoreType.DMA,
            pltpu.SemaphoreType.DMA,
            pltpu.SemaphoreType.DMA,
            pltpu.SemaphoreType.DMA,
            pltpu.SemaphoreType.DMA,
            pltpu.SemaphoreType.DMA,
        ],
        compiler_params=pltpu.CompilerParams(use_tc_tiling_on_sc=False),
    )
    return f(eflat, p)


def _tc_encode(text_f, vis_f, W_t, b_t, W_v, b_v, W_g0):
    N, T = text_f.shape
    V = vis_f.shape[1]
    H = W_t.shape[1]
    RB = 1000
    assert N % RB == 0

    def body(t_ref, v_ref, wt_ref, bt_ref, wv_ref, bv_ref, wg_ref, o_ref):
        ht = jnp.maximum(
            jnp.dot(t_ref[...], wt_ref[...], preferred_element_type=jnp.float32)
            + bt_ref[...], 0.0)
        hv = jnp.maximum(
            jnp.dot(v_ref[...], wv_ref[...], preferred_element_type=jnp.float32)
            + bv_ref[...], 0.0)
        h = jnp.concatenate([ht, hv], axis=1)
        o_ref[...] = jnp.dot(h, wg_ref[...], preferred_element_type=jnp.float32)

    return pl.pallas_call(
        body,
        grid=(N // RB,),
        in_specs=[
            pl.BlockSpec((RB, T), lambda i: (i, 0)),
            pl.BlockSpec((RB, V), lambda i: (i, 0)),
            pl.BlockSpec((T, H), lambda i: (0, 0)),
            pl.BlockSpec((1, H), lambda i: (0, 0)),
            pl.BlockSpec((V, H), lambda i: (0, 0)),
            pl.BlockSpec((1, H), lambda i: (0, 0)),
            pl.BlockSpec((2 * H, H), lambda i: (0, 0)),
        ],
        out_specs=pl.BlockSpec((RB, H), lambda i: (i, 0)),
        out_shape=jax.ShapeDtypeStruct((N, H), jnp.float32),
    )(text_f, vis_f, W_t, b_t, W_v, b_v, W_g0)


def _tc_prescale(z, dout, N):
    H = z.shape[1]
    RB = 1000
    assert N % RB == 0

    def body(z_ref, do_ref, o_ref):
        d = do_ref[0] + do_ref[1]
        inv = 1.0 / jnp.sqrt(jnp.maximum(d, 1.0))
        o_ref[...] = z_ref[...] * inv

    return pl.pallas_call(
        body,
        grid=(N // RB,),
        in_specs=[
            pl.BlockSpec((RB, H), lambda i: (i, 0)),
            pl.BlockSpec((NC, RB, 1), lambda i: (0, i, 0)),
        ],
        out_specs=pl.BlockSpec((RB, H), lambda i: (i, 0)),
        out_shape=jax.ShapeDtypeStruct((N, H), jnp.float32),
    )(z, dout)


def _tc_mid(agg, din, dout, b_g0, W_g1, N):
    H = W_g1.shape[0]
    RB = 1000
    assert N % RB == 0

    def body(a_ref, di_ref, do_ref, b_ref, w_ref, o_ref):
        a = a_ref[0] + a_ref[1]
        inv_in = 1.0 / jnp.sqrt(jnp.maximum(di_ref[0] + di_ref[1], 1.0))
        h = jnp.maximum(a * inv_in + b_ref[...], 0.0)
        z = jnp.dot(h, w_ref[...], preferred_element_type=jnp.float32)
        inv_out = 1.0 / jnp.sqrt(jnp.maximum(do_ref[0] + do_ref[1], 1.0))
        o_ref[...] = z * inv_out

    return pl.pallas_call(
        body,
        grid=(N // RB,),
        in_specs=[
            pl.BlockSpec((NC, RB, H), lambda i: (0, i, 0)),
            pl.BlockSpec((NC, RB, 1), lambda i: (0, i, 0)),
            pl.BlockSpec((NC, RB, 1), lambda i: (0, i, 0)),
            pl.BlockSpec((1, H), lambda i: (0, 0)),
            pl.BlockSpec((H, H), lambda i: (0, 0)),
        ],
        out_specs=pl.BlockSpec((RB, H), lambda i: (i, 0)),
        out_shape=jax.ShapeDtypeStruct((N, H), jnp.float32),
    )(agg, din, dout, b_g0, W_g1)


def _tc_head(agg, din, b_g1, W_head, b_head, N):
    H, C = W_head.shape
    RB = 1000
    assert N % RB == 0

    def body(a_ref, di_ref, b_ref, w_ref, bh_ref, o_ref):
        a = a_ref[0] + a_ref[1]
        inv_in = 1.0 / jnp.sqrt(jnp.maximum(di_ref[0] + di_ref[1], 1.0))
        h = jnp.maximum(a * inv_in + b_ref[...], 0.0)
        o_ref[...] = (
            jnp.dot(h, w_ref[...], preferred_element_type=jnp.float32) + bh_ref[...]
        )

    return pl.pallas_call(
        body,
        grid=(N // RB,),
        in_specs=[
            pl.BlockSpec((NC, RB, H), lambda i: (0, i, 0)),
            pl.BlockSpec((NC, RB, 1), lambda i: (0, i, 0)),
            pl.BlockSpec((1, H), lambda i: (0, 0)),
            pl.BlockSpec((H, C), lambda i: (0, 0)),
            pl.BlockSpec((1, C), lambda i: (0, 0)),
        ],
        out_specs=pl.BlockSpec((RB, C), lambda i: (i, 0)),
        out_shape=jax.ShapeDtypeStruct((N, C), jnp.float32),
    )(agg, din, b_g1, W_head, b_head)


def kernel(edge_index, text_f, vis_f, W_t, b_t, W_v, b_v, W_g0, b_g0, W_g1, b_g1,
           W_head, b_head):
    N = text_f.shape[0]
    E = edge_index.shape[1]
    n_pad = -(-N // (NS * 64)) * (NS * 64)

    deg = _sc_degrees(edge_index, n_pad)
    z0 = _tc_encode(text_f, vis_f, W_t, b_t.reshape(1, -1), W_v,
                    b_v.reshape(1, -1), W_g0)
    din = deg[:, 0, :, None]
    dout = deg[:, 1, :, None]
    p0 = _tc_prescale(z0, dout, N)
    agg0 = _sc_aggregate(edge_index, p0, n_pad)
    p1 = _tc_mid(agg0, din, dout, b_g0.reshape(1, -1), W_g1, N)
    agg1 = _sc_aggregate(edge_index, p1, n_pad)
    out = _tc_head(agg1, din, b_g1.reshape(1, -1), W_head, b_head.reshape(1, -1), N)
    return out


# speedup vs baseline: 36.6251x; 1.1974x over previous
import functools

import jax
import jax.numpy as jnp
from jax import lax
from jax.experimental import pallas as pl
from jax.experimental.pallas import tpu as pltpu
from jax.experimental.pallas import tpu_sc as plsc

NC = 2
NS = 16
NW = NC * NS
LANE = 128


def _mesh():
    return plsc.VectorSubcoreMesh(
        core_axis_name="c", subcore_axis_name="s", num_cores=NC, num_subcores=NS
    )


def _fill(ref, n, value):
    def body(i, _):
        ref[pl.ds(i * 16, 16)] = jnp.full((16,), value, jnp.float32)
        return 0
    lax.fori_loop(0, n // 16, body, 0)


def _sc_degrees(eflat, n_pad):
    E = eflat.shape[1]
    CH = 128
    assert E % NW == 0
    per_tile = E // NW
    full = per_tile // CH
    tail = per_tile % CH
    pairs, odd = divmod(full, 2)
    assert tail % 16 == 0 and per_tile % 8 == 0
    rpt = n_pad // NS

    def body(e_hbm, out_hbm, sbA, dbA, sbB, dbB, sbt, dbt, ones_v, onest_v, zer_v,
             din_sp, dout_sp, semA, semB, semC, semD):
        c = lax.axis_index("c")
        s = lax.axis_index("s")
        wid = c * NS + s
        base0 = wid * per_tile

        _fill(zer_v, rpt, 0.0)
        _fill(ones_v, CH, 1.0)
        if tail:
            _fill(onest_v, tail, 1.0)
        pltpu.sync_copy(zer_v, din_sp.at[pl.ds(s * rpt, rpt)])
        pltpu.sync_copy(zer_v, dout_sp.at[pl.ds(s * rpt, rpt)])
        plsc.subcore_barrier()

        def pbody(i, _):
            b0 = base0 + (2 * i) * CH
            b1 = b0 + CH
            dA = pltpu.async_copy(e_hbm.at[0, pl.ds(b0, CH)], sbA, semA)
            dB = pltpu.async_copy(e_hbm.at[1, pl.ds(b0, CH)], dbA, semB)
            dC = pltpu.async_copy(e_hbm.at[0, pl.ds(b1, CH)], sbB, semC)
            dD = pltpu.async_copy(e_hbm.at[1, pl.ds(b1, CH)], dbB, semD)
            dA.wait()
            pltpu.sync_copy(ones_v, dout_sp.at[sbA], add=True)
            dB.wait()
            pltpu.sync_copy(ones_v, din_sp.at[dbA], add=True)
            dC.wait()
            pltpu.sync_copy(ones_v, dout_sp.at[sbB], add=True)
            dD.wait()
            pltpu.sync_copy(ones_v, din_sp.at[dbB], add=True)
            return 0

        lax.fori_loop(0, pairs, pbody, 0)

        if odd:
            b0 = base0 + (pairs * 2) * CH
            pltpu.sync_copy(e_hbm.at[0, pl.ds(b0, CH)], sbA)
            pltpu.sync_copy(ones_v, dout_sp.at[sbA], add=True)
            pltpu.sync_copy(e_hbm.at[1, pl.ds(b0, CH)], dbA)
            pltpu.sync_copy(ones_v, din_sp.at[dbA], add=True)

        if tail:
            bt = base0 + full * CH
            pltpu.sync_copy(e_hbm.at[0, pl.ds(bt, tail)], sbt)
            pltpu.sync_copy(onest_v, dout_sp.at[sbt], add=True)
            pltpu.sync_copy(e_hbm.at[1, pl.ds(bt, tail)], dbt)
            pltpu.sync_copy(onest_v, din_sp.at[dbt], add=True)

        plsc.subcore_barrier()

        pltpu.sync_copy(din_sp.at[pl.ds(s * rpt, rpt)],
                        out_hbm.at[c, 0, pl.ds(s * rpt, rpt)])
        pltpu.sync_copy(dout_sp.at[pl.ds(s * rpt, rpt)],
                        out_hbm.at[c, 1, pl.ds(s * rpt, rpt)])

    f = pl.kernel(
        body,
        out_type=jax.ShapeDtypeStruct((NC, 2, n_pad), jnp.float32),
        mesh=_mesh(),
        scratch_types=[
            pltpu.VMEM((CH,), jnp.int32),
            pltpu.VMEM((CH,), jnp.int32),
            pltpu.VMEM((CH,), jnp.int32),
            pltpu.VMEM((CH,), jnp.int32),
            pltpu.VMEM((max(tail, 16),), jnp.int32),
            pltpu.VMEM((max(tail, 16),), jnp.int32),
            pltpu.VMEM((CH,), jnp.float32),
            pltpu.VMEM((max(tail, 16),), jnp.float32),
            pltpu.VMEM((rpt,), jnp.float32),
            pltpu.VMEM_SHARED((n_pad,), jnp.float32),
            pltpu.VMEM_SHARED((n_pad,), jnp.float32),
            pltpu.SemaphoreType.DMA,
            pltpu.SemaphoreType.DMA,
            pltpu.SemaphoreType.DMA,
            pltpu.SemaphoreType.DMA,
        ],
        compiler_params=pltpu.CompilerParams(use_tc_tiling_on_sc=False),
    )
    return f(eflat)


def _sc_aggregate(eflat, p, n_pad):
    E = eflat.shape[1]
    H = p.shape[1]
    CH = 128
    assert E % NW == 0
    per_tile = E // NW
    full = per_tile // CH
    tail = per_tile % CH
    triples, rem = divmod(full, 3)
    assert tail % 8 == 0 and per_tile % 8 == 0
    rpt = n_pad // NS
    ZR = 64
    assert rpt % ZR == 0

    def body(e_hbm, p_hbm, out_hbm, sbig, dbA, dbB, dbC, dbt, rowsA, rowsB,
             rowsC, rowst, zer_v, acc_sp, semDA, semDB, semDC, semGA, semGB,
             semGC, semSA, semSB, semSC):
        c = lax.axis_index("c")
        s = lax.axis_index("s")
        wid = c * NS + s
        base0 = wid * per_tile

        def zfill(i, _):
            zer_v[i, pl.ds(0, 16)] = jnp.zeros((16,), jnp.float32)
            zer_v[i, pl.ds(16, 16)] = jnp.zeros((16,), jnp.float32)
            zer_v[i, pl.ds(32, 16)] = jnp.zeros((16,), jnp.float32)
            zer_v[i, pl.ds(48, 16)] = jnp.zeros((16,), jnp.float32)
            return 0
        lax.fori_loop(0, ZR, zfill, 0)

        def zcopy(i, _):
            pltpu.sync_copy(zer_v, acc_sp.at[pl.ds(s * rpt + i * ZR, ZR)])
            return 0
        lax.fori_loop(0, rpt // ZR, zcopy, 0)

        pltpu.sync_copy(e_hbm.at[0, pl.ds(base0, per_tile)], sbig)
        plsc.subcore_barrier()

        bufs = ((dbA, rowsA, semDA, semGA, semSA),
                (dbB, rowsB, semDB, semGB, semSB),
                (dbC, rowsC, semDC, semGC, semSC))

        def tbody(i, _):
            descs = []
            for k, (db, rows, semD, semG, semS) in enumerate(bufs):
                b0 = (3 * i + k) * CH
                @pl.when(i > 0)
                def _(db=db, rows=rows, semS=semS):
                    pltpu.make_async_copy(rows, acc_sp.at[db], semS).wait()
                descs.append((
                    pltpu.async_copy(e_hbm.at[1, pl.ds(base0 + b0, CH)], db, semD),
                    pltpu.async_copy(p_hbm.at[sbig.at[pl.ds(b0, CH)]], rows, semG),
                ))
            for (d, g), (db, rows, _, _, semS) in zip(descs, bufs):
                d.wait()
                g.wait()
                pltpu.async_copy(rows, acc_sp.at[db], semS, add=True)
            return 0

        lax.fori_loop(0, triples, tbody, 0)
        if triples > 0:
            for db, rows, _, _, semS in bufs:
                pltpu.make_async_copy(rows, acc_sp.at[db], semS).wait()

        for r in range(rem):
            b0 = (triples * 3 + r) * CH
            pltpu.sync_copy(e_hbm.at[1, pl.ds(base0 + b0, CH)], dbA)
            pltpu.async_copy(p_hbm.at[sbig.at[pl.ds(b0, CH)]], rowsA, semGA).wait()
            pltpu.sync_copy(rowsA, acc_sp.at[dbA], add=True)

        if tail:
            bt = full * CH
            pltpu.sync_copy(e_hbm.at[1, pl.ds(base0 + bt, tail)], dbt)
            pltpu.async_copy(p_hbm.at[sbig.at[pl.ds(bt, tail)]], rowst, semGA).wait()
            pltpu.sync_copy(rowst, acc_sp.at[dbt], add=True)

        plsc.subcore_barrier()
        pltpu.sync_copy(acc_sp.at[pl.ds(s * rpt, rpt)],
                        out_hbm.at[pl.ds(s * rpt, rpt), pl.ds(c * H, H)])

    f = pl.kernel(
        body,
        out_type=jax.ShapeDtypeStruct((n_pad, NC * H), jnp.float32),
        mesh=_mesh(),
        scratch_types=[
            pltpu.VMEM((per_tile,), jnp.int32),
            pltpu.VMEM((CH,), jnp.int32),
            pltpu.VMEM((CH,), jnp.int32),
            pltpu.VMEM((CH,), jnp.int32),
            pltpu.VMEM((max(tail, 8),), jnp.int32),
            pltpu.VMEM((CH, H), jnp.float32),
            pltpu.VMEM((CH, H), jnp.float32),
            pltpu.VMEM((CH, H), jnp.float32),
            pltpu.VMEM((max(tail, 8), H), jnp.float32),
            pltpu.VMEM((ZR, H), jnp.float32),
            pltpu.VMEM_SHARED((n_pad, H), jnp.float32),
            pltpu.SemaphoreType.DMA,
            pltpu.SemaphoreType.DMA,
            pltpu.SemaphoreType.DMA,
            pltpu.SemaphoreType.DMA,
            pltpu.SemaphoreType.DMA,
            pltpu.SemaphoreType.DMA,
            pltpu.SemaphoreType.DMA,
            pltpu.SemaphoreType.DMA,
            pltpu.SemaphoreType.DMA,
        ],
        compiler_params=pltpu.CompilerParams(use_tc_tiling_on_sc=False),
    )
    return f(eflat, p)


def _tc_encode(text_f, vis_f, W_t, b_t, W_v, b_v, W_g0):
    N, T = text_f.shape
    V = vis_f.shape[1]
    H = W_t.shape[1]
    RB = 1000
    assert N % RB == 0

    def body(t_ref, v_ref, wt_ref, bt_ref, wv_ref, bv_ref, wg_ref, o_ref):
        ht = jnp.maximum(
            jnp.dot(t_ref[...], wt_ref[...], preferred_element_type=jnp.float32)
            + bt_ref[...], 0.0)
        hv = jnp.maximum(
            jnp.dot(v_ref[...], wv_ref[...], preferred_element_type=jnp.float32)
            + bv_ref[...], 0.0)
        h = jnp.concatenate([ht, hv], axis=1)
        o_ref[...] = jnp.dot(h, wg_ref[...], preferred_element_type=jnp.float32)

    return pl.pallas_call(
        body,
        grid=(N // RB,),
        in_specs=[
            pl.BlockSpec((RB, T), lambda i: (i, 0)),
            pl.BlockSpec((RB, V), lambda i: (i, 0)),
            pl.BlockSpec((T, H), lambda i: (0, 0)),
            pl.BlockSpec((1, H), lambda i: (0, 0)),
            pl.BlockSpec((V, H), lambda i: (0, 0)),
            pl.BlockSpec((1, H), lambda i: (0, 0)),
            pl.BlockSpec((2 * H, H), lambda i: (0, 0)),
        ],
        out_specs=pl.BlockSpec((RB, H), lambda i: (i, 0)),
        out_shape=jax.ShapeDtypeStruct((N, H), jnp.float32),
    )(text_f, vis_f, W_t, b_t, W_v, b_v, W_g0)


def _tc_prescale(z, deg, N, n_pad):
    H = z.shape[1]

    def body(z_ref, dg_ref, o_ref, iv_ref):
        d = dg_ref[0] + dg_ref[1]
        inv = 1.0 / jnp.sqrt(jnp.maximum(d, 1.0))
        invt = jnp.transpose(inv, (1, 0))
        iv_ref[...] = invt
        o_ref[...] = z_ref[...] * invt[:N, 1:2]

    return pl.pallas_call(
        body,
        out_shape=(
            jax.ShapeDtypeStruct((N, H), jnp.float32),
            jax.ShapeDtypeStruct((n_pad, 2), jnp.float32),
        ),
    )(z, deg)


def _tc_mid(agg, invs, b_g0, W_g1, N):
    H = W_g1.shape[0]
    RB = 1000
    assert N % RB == 0

    def body(a_ref, iv_ref, b_ref, w_ref, o_ref):
        a = a_ref[:, :H] + a_ref[:, H:]
        iv = iv_ref[...]
        h = jnp.maximum(a * iv[:, 0:1] + b_ref[...], 0.0)
        z = jnp.dot(h, w_ref[...], preferred_element_type=jnp.float32)
        o_ref[...] = z * iv[:, 1:2]

    return pl.pallas_call(
        body,
        grid=(N // RB,),
        in_specs=[
            pl.BlockSpec((RB, NC * H), lambda i: (i, 0)),
            pl.BlockSpec((RB, 2), lambda i: (i, 0)),
            pl.BlockSpec((1, H), lambda i: (0, 0)),
            pl.BlockSpec((H, H), lambda i: (0, 0)),
        ],
        out_specs=pl.BlockSpec((RB, H), lambda i: (i, 0)),
        out_shape=jax.ShapeDtypeStruct((N, H), jnp.float32),
    )(agg, invs, b_g0, W_g1)


def _tc_head(agg, invs, b_g1, W_head, b_head, N):
    H, C = W_head.shape
    RB = 1000
    assert N % RB == 0

    def body(a_ref, iv_ref, b_ref, w_ref, bh_ref, o_ref):
        a = a_ref[:, :H] + a_ref[:, H:]
        iv = iv_ref[...]
        h = jnp.maximum(a * iv[:, 0:1] + b_ref[...], 0.0)
        o_ref[...] = (
            jnp.dot(h, w_ref[...], preferred_element_type=jnp.float32) + bh_ref[...]
        )

    return pl.pallas_call(
        body,
        grid=(N // RB,),
        in_specs=[
            pl.BlockSpec((RB, NC * H), lambda i: (i, 0)),
            pl.BlockSpec((RB, 2), lambda i: (i, 0)),
            pl.BlockSpec((1, H), lambda i: (0, 0)),
            pl.BlockSpec((H, C), lambda i: (0, 0)),
            pl.BlockSpec((1, C), lambda i: (0, 0)),
        ],
        out_specs=pl.BlockSpec((RB, C), lambda i: (i, 0)),
        out_shape=jax.ShapeDtypeStruct((N, C), jnp.float32),
    )(agg, invs, b_g1, W_head, b_head)


def kernel(edge_index, text_f, vis_f, W_t, b_t, W_v, b_v, W_g0, b_g0, W_g1, b_g1,
           W_head, b_head):
    N = text_f.shape[0]
    E = edge_index.shape[1]
    n_pad = -(-N // (NS * 64)) * (NS * 64)

    deg = _sc_degrees(edge_index, n_pad)
    z0 = _tc_encode(text_f, vis_f, W_t, b_t.reshape(1, -1), W_v,
                    b_v.reshape(1, -1), W_g0)
    p0, invs = _tc_prescale(z0, deg, N, n_pad)
    agg0 = _sc_aggregate(edge_index, p0, n_pad)
    p1 = _tc_mid(agg0, invs, b_g0.reshape(1, -1), W_g1, N)
    agg1 = _sc_aggregate(edge_index, p1, n_pad)
    out = _tc_head(agg1, invs, b_g1.reshape(1, -1), W_head, b_head.reshape(1, -1), N)
    return out
